# Initial kernel scaffold; baseline (speedup 1.0000x reference)
import jax
import jax.numpy as jnp
from jax.experimental import pallas as pl


def kernel(x, edge_index, W, att_src, att_dst, bias):
    raise NotImplementedError("write your pallas kernel here")


# speedup vs baseline: 25.6039x; 25.6039x over previous
import functools

import jax
import jax.numpy as jnp
from jax import lax
from jax.experimental import pallas as pl
from jax.experimental.pallas import tpu as pltpu
from jax.experimental.pallas import tpu_sc as plsc

N = 10000
E = 320000
D = 128

NC = 2
NS = 16
NW = NC * NS
K = 64
EPW = 9984
NCHUNK = EPW // K
NCHUNK_LAST = (E - (NW - 1) * EPW) // K
NP_ = 10240
RPT = NP_ // NS


def _p1_body(x_ref, w_ref, as_ref, ad_ref, h_ref, asrc_ref, adst_ref):
    h = jnp.dot(x_ref[...], w_ref[...], preferred_element_type=jnp.float32)
    h_ref[...] = h
    asrc_ref[...] = jnp.dot(h, as_ref[...], preferred_element_type=jnp.float32)
    adst_ref[...] = jnp.dot(h, ad_ref[...], preferred_element_type=jnp.float32)


def _phase1(x, W, att_src, att_dst):
    return pl.pallas_call(
        _p1_body,
        out_shape=[
            jax.ShapeDtypeStruct((N, D), jnp.float32),
            jax.ShapeDtypeStruct((N,), jnp.float32),
            jax.ShapeDtypeStruct((N,), jnp.float32),
        ],
    )(x, W, att_src, att_dst)


_mesh = plsc.VectorSubcoreMesh(core_axis_name="c", subcore_axis_name="s")


@functools.partial(
    pl.kernel,
    out_type=[
        jax.ShapeDtypeStruct((NC, NP_, D), jnp.float32),
        jax.ShapeDtypeStruct((NW, NP_), jnp.float32),
    ],
    mesh=_mesh,
    compiler_params=pltpu.CompilerParams(needs_layout_passes=False),
    scratch_types=[
        pltpu.VMEM((N,), jnp.float32),
        pltpu.VMEM((N,), jnp.float32),
        pltpu.VMEM((NP_,), jnp.float32),
        pltpu.VMEM((K,), jnp.int32),
        pltpu.VMEM((K,), jnp.int32),
        pltpu.VMEM((K, D), jnp.float32),
        pltpu.VMEM_SHARED((NP_, D), jnp.float32),
        pltpu.SemaphoreType.DMA,
    ],
)
def _phase2(h_hbm, asrc_hbm, adst_hbm, src_hbm, dst_hbm, num_hbm, den_hbm,
            asrc_v, adst_v, den_v, sidx_v, didx_v, rows_v,
            num_sh, sem):
    c = lax.axis_index("c")
    s = lax.axis_index("s")
    wid = c * NS + s
    zero16 = jnp.zeros((16,), jnp.float32)

    def zrow(i, carry):
        for j in range(D // 16):
            rows_v[i, pl.ds(j * 16, 16)] = zero16
        return carry

    lax.fori_loop(0, K, zrow, 0)

    def zden(i, carry):
        den_v[pl.ds(i * 16, 16)] = zero16
        return carry

    lax.fori_loop(0, NP_ // 16, zden, 0)
    row0 = s * RPT

    def zcp(k, carry):
        pltpu.sync_copy(rows_v, num_sh.at[pl.ds(row0 + k * K, K)])
        return carry

    lax.fori_loop(0, RPT // K, zcp, 0)
    pltpu.sync_copy(asrc_hbm, asrc_v)
    pltpu.sync_copy(adst_hbm, adst_v)
    plsc.subcore_barrier()

    ebase = wid * EPW
    nchunk = jnp.where(wid == NW - 1, NCHUNK_LAST, NCHUNK)

    def chunk(i, carry):
        base = ebase + i * K
        pltpu.sync_copy(src_hbm.at[pl.ds(base, K)], sidx_v)
        pltpu.sync_copy(dst_hbm.at[pl.ds(base, K)], didx_v)
        pltpu.async_copy(h_hbm.at[sidx_v], rows_v, sem).wait()
        for g in range(K // 16):
            sv = sidx_v[pl.ds(g * 16, 16)]
            dv = didx_v[pl.ds(g * 16, 16)]
            e = plsc.load_gather(asrc_v, [sv]) + plsc.load_gather(adst_v, [dv])
            e = jnp.where(e >= 0.0, e, 0.2 * e)
            w = jnp.exp(e)
            plsc.addupdate_scatter(den_v, [dv], w)
            for l in range(16):
                ws = w[l]
                k = g * 16 + l
                for j in range(D // 16):
                    rows_v[k, pl.ds(j * 16, 16)] = (
                        rows_v[k, pl.ds(j * 16, 16)] * ws)
        pltpu.sync_copy(rows_v, num_sh.at[didx_v], add=True)
        return carry

    lax.fori_loop(0, nchunk, chunk, 0)
    plsc.subcore_barrier()
    pltpu.sync_copy(num_sh.at[pl.ds(row0, RPT)], num_hbm.at[c, pl.ds(row0, RPT)])
    pltpu.sync_copy(den_v, den_hbm.at[wid])


def _p3_body(num_ref, den_ref, h_ref, asrc_ref, adst_ref, b_ref, o_ref):
    es = asrc_ref[...] + adst_ref[...]
    es = jnp.where(es >= 0.0, es, 0.2 * es)
    wself = jnp.exp(es)
    num = num_ref[0, :N] + num_ref[1, :N] + wself[:, None] * h_ref[...]
    den = jnp.sum(den_ref[...], axis=0)[:N] + wself
    o_ref[...] = num / den[:, None] + b_ref[...][None, :]


def _phase3(num2, den2, h, a_src, a_dst, bias):
    return pl.pallas_call(
        _p3_body,
        out_shape=jax.ShapeDtypeStruct((N, D), jnp.float32),
    )(num2, den2, h, a_src, a_dst, bias)


def kernel(x, edge_index, W, att_src, att_dst, bias):
    src32 = edge_index[0].astype(jnp.int32)
    dst32 = edge_index[1].astype(jnp.int32)
    h, a_src, a_dst = _phase1(x, W, att_src, att_dst)
    num2, den2 = _phase2(h, a_src, a_dst, src32, dst32)
    out = _phase3(num2, den2, h, a_src, a_dst, bias)
    return out


# speedup vs baseline: 35.4285x; 1.3837x over previous
import functools

import jax
import jax.numpy as jnp
from jax import lax
from jax.experimental import pallas as pl
from jax.experimental.pallas import tpu as pltpu
from jax.experimental.pallas import tpu_sc as plsc

N = 10000
E = 320000
D = 128

NC = 2
NS = 16
NW = NC * NS
K = 32
EPW = 9984
NCHUNK = EPW // K
NCHUNK_LAST = (E - (NW - 1) * EPW) // K
NP_ = 10240
RPT = NP_ // NS


def _p1_body(x_ref, w_ref, as_ref, ad_ref, h_ref, asrc_ref, adst_ref):
    h = jnp.dot(x_ref[...], w_ref[...], preferred_element_type=jnp.float32)
    h_ref[...] = h
    asrc_ref[...] = jnp.dot(h, as_ref[...], preferred_element_type=jnp.float32)
    adst_ref[...] = jnp.dot(h, ad_ref[...], preferred_element_type=jnp.float32)


def _phase1(x, W, att_src, att_dst):
    return pl.pallas_call(
        _p1_body,
        out_shape=[
            jax.ShapeDtypeStruct((N, D), jnp.float32),
            jax.ShapeDtypeStruct((N,), jnp.float32),
            jax.ShapeDtypeStruct((N,), jnp.float32),
        ],
    )(x, W, att_src, att_dst)


_mesh = plsc.VectorSubcoreMesh(core_axis_name="c", subcore_axis_name="s")


@functools.partial(
    pl.kernel,
    out_type=[
        jax.ShapeDtypeStruct((NC, NP_, D), jnp.float32),
        jax.ShapeDtypeStruct((NW, NP_), jnp.float32),
    ],
    mesh=_mesh,
    compiler_params=pltpu.CompilerParams(needs_layout_passes=False),
    scratch_types=[
        pltpu.VMEM((N,), jnp.float32),
        pltpu.VMEM((N,), jnp.float32),
        pltpu.VMEM((NP_,), jnp.float32),
        pltpu.VMEM((K,), jnp.int32),
        pltpu.VMEM((K,), jnp.int32),
        pltpu.VMEM((K, D), jnp.float32),
        pltpu.VMEM((K,), jnp.int32),
        pltpu.VMEM((K,), jnp.int32),
        pltpu.VMEM((K, D), jnp.float32),
        pltpu.VMEM_SHARED((NP_, D), jnp.float32),
        pltpu.SemaphoreType.DMA,
        pltpu.SemaphoreType.DMA,
        pltpu.SemaphoreType.DMA,
        pltpu.SemaphoreType.DMA,
    ],
)
def _phase2(h_hbm, asrc_hbm, adst_hbm, src_hbm, dst_hbm, num_hbm, den_hbm,
            asrc_v, adst_v, den_v, sidx_a, didx_a, rows_a,
            sidx_b, didx_b, rows_b,
            num_sh, semg_a, semg_b, semi_a, semi_b):
    c = lax.axis_index("c")
    s = lax.axis_index("s")
    wid = c * NS + s
    zero16 = jnp.zeros((16,), jnp.float32)

    def zrow(i, carry):
        for j in range(D // 16):
            rows_a[i, pl.ds(j * 16, 16)] = zero16
        return carry

    lax.fori_loop(0, K, zrow, 0)

    def zden(i, carry):
        den_v[pl.ds(i * 16, 16)] = zero16
        return carry

    lax.fori_loop(0, NP_ // 16, zden, 0)
    row0 = s * RPT

    def zcp(k, carry):
        pltpu.sync_copy(rows_a, num_sh.at[pl.ds(row0 + k * K, K)])
        return carry

    lax.fori_loop(0, RPT // K, zcp, 0)
    pltpu.sync_copy(asrc_hbm, asrc_v)
    pltpu.sync_copy(adst_hbm, adst_v)
    plsc.subcore_barrier()

    ebase = wid * EPW
    nchunk = jnp.where(wid == NW - 1, NCHUNK_LAST, NCHUNK)

    def scale_and_scatter(sidx_v, didx_v, rows_v):
        for g in range(K // 16):
            sv = sidx_v[pl.ds(g * 16, 16)]
            dv = didx_v[pl.ds(g * 16, 16)]
            e = plsc.load_gather(asrc_v, [sv]) + plsc.load_gather(adst_v, [dv])
            e = jnp.where(e >= 0.0, e, 0.2 * e)
            w = jnp.exp(e)
            plsc.addupdate_scatter(den_v, [dv], w)
            for l in range(16):
                ws = w[l]
                k = g * 16 + l
                for j in range(D // 16):
                    rows_v[k, pl.ds(j * 16, 16)] = (
                        rows_v[k, pl.ds(j * 16, 16)] * ws)
        pltpu.sync_copy(rows_v, num_sh.at[didx_v], add=True)

    pltpu.sync_copy(src_hbm.at[pl.ds(ebase, K)], sidx_a)
    pltpu.sync_copy(dst_hbm.at[pl.ds(ebase, K)], didx_a)
    pltpu.async_copy(h_hbm.at[sidx_a], rows_a, semg_a)
    pltpu.async_copy(src_hbm.at[pl.ds(ebase + K, K)], sidx_b, semi_b)
    pltpu.async_copy(dst_hbm.at[pl.ds(ebase + K, K)], didx_b, semi_b)

    def pair(p, carry):
        c0 = 2 * p
        pf_a = jnp.minimum(c0 + 2, nchunk - 2)
        pf_b = jnp.minimum(c0 + 3, nchunk - 1)
        pltpu.make_async_copy(h_hbm.at[pl.ds(0, K)], rows_a, semg_a).wait()
        pltpu.make_async_copy(src_hbm.at[pl.ds(0, K)], sidx_b, semi_b).wait()
        pltpu.make_async_copy(dst_hbm.at[pl.ds(0, K)], didx_b, semi_b).wait()
        gath_b = pltpu.async_copy(h_hbm.at[sidx_b], rows_b, semg_b)
        scale_and_scatter(sidx_a, didx_a, rows_a)
        ld_sa = pltpu.async_copy(src_hbm.at[pl.ds(ebase + pf_a * K, K)],
                                 sidx_a, semi_a)
        ld_da = pltpu.async_copy(dst_hbm.at[pl.ds(ebase + pf_a * K, K)],
                                 didx_a, semi_a)
        gath_b.wait()
        ld_sa.wait()
        ld_da.wait()
        pltpu.async_copy(h_hbm.at[sidx_a], rows_a, semg_a)
        scale_and_scatter(sidx_b, didx_b, rows_b)
        pltpu.async_copy(src_hbm.at[pl.ds(ebase + pf_b * K, K)],
                         sidx_b, semi_b)
        pltpu.async_copy(dst_hbm.at[pl.ds(ebase + pf_b * K, K)],
                         didx_b, semi_b)
        return carry

    lax.fori_loop(0, nchunk // 2, pair, 0)
    pltpu.make_async_copy(h_hbm.at[pl.ds(0, K)], rows_a, semg_a).wait()
    pltpu.make_async_copy(src_hbm.at[pl.ds(0, K)], sidx_b, semi_b).wait()
    pltpu.make_async_copy(dst_hbm.at[pl.ds(0, K)], didx_b, semi_b).wait()
    plsc.subcore_barrier()
    pltpu.sync_copy(num_sh.at[pl.ds(row0, RPT)], num_hbm.at[c, pl.ds(row0, RPT)])
    pltpu.sync_copy(den_v, den_hbm.at[wid])


def _p3_body(num_ref, den_ref, h_ref, asrc_ref, adst_ref, b_ref, o_ref):
    es = asrc_ref[...] + adst_ref[...]
    es = jnp.where(es >= 0.0, es, 0.2 * es)
    wself = jnp.exp(es)
    num = num_ref[0, :N] + num_ref[1, :N] + wself[:, None] * h_ref[...]
    den = jnp.sum(den_ref[...], axis=0)[:N] + wself
    o_ref[...] = num / den[:, None] + b_ref[...][None, :]


def _phase3(num2, den2, h, a_src, a_dst, bias):
    return pl.pallas_call(
        _p3_body,
        out_shape=jax.ShapeDtypeStruct((N, D), jnp.float32),
    )(num2, den2, h, a_src, a_dst, bias)


def kernel(x, edge_index, W, att_src, att_dst, bias):
    src32 = edge_index[0].astype(jnp.int32)
    dst32 = edge_index[1].astype(jnp.int32)
    h, a_src, a_dst = _phase1(x, W, att_src, att_dst)
    num2, den2 = _phase2(h, a_src, a_dst, src32, dst32)
    out = _phase3(num2, den2, h, a_src, a_dst, bias)
    return out


# speedup vs baseline: 45.5219x; 1.2849x over previous
import functools

import jax
import jax.numpy as jnp
from jax import lax
from jax.experimental import pallas as pl
from jax.experimental.pallas import tpu as pltpu
from jax.experimental.pallas import tpu_sc as plsc

N = 10000
E = 320000
D = 128

NC = 2
NS = 16
NW = NC * NS
K = 64
EPW = 9984
NCHUNK = EPW // K
NCHUNK_LAST = (E - (NW - 1) * EPW) // K
NP_ = 10240
RPT = NP_ // NS


def _p1_body(x_ref, w_ref, as_ref, ad_ref, h_ref, asrc_ref, adst_ref):
    h = jnp.dot(x_ref[...], w_ref[...], preferred_element_type=jnp.float32)
    h_ref[...] = h
    asrc_ref[...] = jnp.dot(h, as_ref[...], preferred_element_type=jnp.float32)
    adst_ref[...] = jnp.dot(h, ad_ref[...], preferred_element_type=jnp.float32)


def _phase1(x, W, att_src, att_dst):
    return pl.pallas_call(
        _p1_body,
        out_shape=[
            jax.ShapeDtypeStruct((N, D), jnp.float32),
            jax.ShapeDtypeStruct((N,), jnp.float32),
            jax.ShapeDtypeStruct((N,), jnp.float32),
        ],
    )(x, W, att_src, att_dst)


_mesh = plsc.VectorSubcoreMesh(core_axis_name="c", subcore_axis_name="s")


@functools.partial(
    pl.kernel,
    out_type=[
        jax.ShapeDtypeStruct((NC, NP_, D), jnp.float32),
        jax.ShapeDtypeStruct((NW, NP_), jnp.float32),
    ],
    mesh=_mesh,
    compiler_params=pltpu.CompilerParams(needs_layout_passes=False),
    scratch_types=[
        pltpu.VMEM((N,), jnp.float32),
        pltpu.VMEM((N,), jnp.float32),
        pltpu.VMEM((NP_,), jnp.float32),
        pltpu.VMEM((K,), jnp.int32),
        pltpu.VMEM((K,), jnp.int32),
        pltpu.VMEM((K, D), jnp.float32),
        pltpu.VMEM((K,), jnp.int32),
        pltpu.VMEM((K,), jnp.int32),
        pltpu.VMEM((K, D), jnp.float32),
        pltpu.VMEM_SHARED((NP_, D), jnp.float32),
        pltpu.SemaphoreType.DMA,
        pltpu.SemaphoreType.DMA,
        pltpu.SemaphoreType.DMA,
        pltpu.SemaphoreType.DMA,
    ],
)
def _phase2(h_hbm, asrc_hbm, adst_hbm, src_hbm, dst_hbm, num_hbm, den_hbm,
            asrc_v, adst_v, den_v, sidx_a, didx_a, rows_a,
            sidx_b, didx_b, rows_b,
            num_sh, semg_a, semg_b, semi_a, semi_b):
    c = lax.axis_index("c")
    s = lax.axis_index("s")
    wid = c * NS + s
    zero16 = jnp.zeros((16,), jnp.float32)

    def zrow(i, carry):
        for j in range(D // 16):
            rows_a[i, pl.ds(j * 16, 16)] = zero16
        return carry

    lax.fori_loop(0, K, zrow, 0)

    def zden(i, carry):
        den_v[pl.ds(i * 16, 16)] = zero16
        return carry

    lax.fori_loop(0, NP_ // 16, zden, 0)
    row0 = s * RPT

    def zcp(k, carry):
        pltpu.sync_copy(rows_a, num_sh.at[pl.ds(row0 + k * K, K)])
        return carry

    lax.fori_loop(0, RPT // K, zcp, 0)
    pltpu.sync_copy(asrc_hbm, asrc_v)
    pltpu.sync_copy(adst_hbm, adst_v)
    plsc.subcore_barrier()

    ebase = wid * EPW
    nchunk = jnp.where(wid == NW - 1, NCHUNK_LAST, NCHUNK)

    def scale_and_scatter(sidx_v, didx_v, rows_v):
        for g in range(K // 16):
            sv = sidx_v[pl.ds(g * 16, 16)]
            dv = didx_v[pl.ds(g * 16, 16)]
            e = plsc.load_gather(asrc_v, [sv]) + plsc.load_gather(adst_v, [dv])
            e = jnp.where(e >= 0.0, e, 0.2 * e)
            w = jnp.exp(e)
            plsc.addupdate_scatter(den_v, [dv], w)
            for l in range(16):
                ws = w[l]
                k = g * 16 + l
                for j in range(D // 16):
                    rows_v[k, pl.ds(j * 16, 16)] = (
                        rows_v[k, pl.ds(j * 16, 16)] * ws)
        pltpu.sync_copy(rows_v, num_sh.at[didx_v], add=True)

    pltpu.sync_copy(src_hbm.at[pl.ds(ebase, K)], sidx_a)
    pltpu.sync_copy(dst_hbm.at[pl.ds(ebase, K)], didx_a)
    pltpu.async_copy(h_hbm.at[sidx_a], rows_a, semg_a)
    pltpu.async_copy(src_hbm.at[pl.ds(ebase + K, K)], sidx_b, semi_b)
    pltpu.async_copy(dst_hbm.at[pl.ds(ebase + K, K)], didx_b, semi_b)

    def pair(p, carry):
        c0 = 2 * p
        pf_a = jnp.minimum(c0 + 2, nchunk - 2)
        pf_b = jnp.minimum(c0 + 3, nchunk - 1)
        pltpu.make_async_copy(h_hbm.at[pl.ds(0, K)], rows_a, semg_a).wait()
        pltpu.make_async_copy(src_hbm.at[pl.ds(0, K)], sidx_b, semi_b).wait()
        pltpu.make_async_copy(dst_hbm.at[pl.ds(0, K)], didx_b, semi_b).wait()
        gath_b = pltpu.async_copy(h_hbm.at[sidx_b], rows_b, semg_b)
        scale_and_scatter(sidx_a, didx_a, rows_a)
        ld_sa = pltpu.async_copy(src_hbm.at[pl.ds(ebase + pf_a * K, K)],
                                 sidx_a, semi_a)
        ld_da = pltpu.async_copy(dst_hbm.at[pl.ds(ebase + pf_a * K, K)],
                                 didx_a, semi_a)
        gath_b.wait()
        ld_sa.wait()
        ld_da.wait()
        pltpu.async_copy(h_hbm.at[sidx_a], rows_a, semg_a)
        scale_and_scatter(sidx_b, didx_b, rows_b)
        pltpu.async_copy(src_hbm.at[pl.ds(ebase + pf_b * K, K)],
                         sidx_b, semi_b)
        pltpu.async_copy(dst_hbm.at[pl.ds(ebase + pf_b * K, K)],
                         didx_b, semi_b)
        return carry

    lax.fori_loop(0, nchunk // 2, pair, 0)
    pltpu.make_async_copy(h_hbm.at[pl.ds(0, K)], rows_a, semg_a).wait()
    pltpu.make_async_copy(src_hbm.at[pl.ds(0, K)], sidx_b, semi_b).wait()
    pltpu.make_async_copy(dst_hbm.at[pl.ds(0, K)], didx_b, semi_b).wait()
    plsc.subcore_barrier()
    pltpu.sync_copy(num_sh.at[pl.ds(row0, RPT)], num_hbm.at[c, pl.ds(row0, RPT)])
    pltpu.sync_copy(den_v, den_hbm.at[wid])


def _p3_body(num_ref, den_ref, h_ref, asrc_ref, adst_ref, b_ref, o_ref):
    es = asrc_ref[...] + adst_ref[...]
    es = jnp.where(es >= 0.0, es, 0.2 * es)
    wself = jnp.exp(es)
    num = num_ref[0, :N] + num_ref[1, :N] + wself[:, None] * h_ref[...]
    den = jnp.sum(den_ref[...], axis=0)[:N] + wself
    o_ref[...] = num / den[:, None] + b_ref[...][None, :]


def _phase3(num2, den2, h, a_src, a_dst, bias):
    return pl.pallas_call(
        _p3_body,
        out_shape=jax.ShapeDtypeStruct((N, D), jnp.float32),
    )(num2, den2, h, a_src, a_dst, bias)


def kernel(x, edge_index, W, att_src, att_dst, bias):
    src32 = edge_index[0].astype(jnp.int32)
    dst32 = edge_index[1].astype(jnp.int32)
    h, a_src, a_dst = _phase1(x, W, att_src, att_dst)
    num2, den2 = _phase2(h, a_src, a_dst, src32, dst32)
    out = _phase3(num2, den2, h, a_src, a_dst, bias)
    return out
